# baseline (speedup 1.0000x reference)
import functools

import jax
import jax.numpy as jnp
from jax import lax
from jax.experimental import pallas as pl
from jax.experimental.pallas import tpu as pltpu
from jax.experimental.pallas import tpu_sc as plsc

E = 8
TOP_K = 2
D = 1024
DFF = 1024
SH_DFF = 2048
T = 2048
A = T * TOP_K

TILE = 256
NT = A // TILE + E
NPAD = NT * TILE

BT = 256

_DEF = jax.lax.Precision.DEFAULT
_F32 = jnp.float32


def _excl_cumsum_rows(o, cb=256):
    n = o.shape[0]
    ii = lax.broadcasted_iota(jnp.int32, (cb, cb), 0)
    jj = lax.broadcasted_iota(jnp.int32, (cb, cb), 1)
    slt = (jj < ii).astype(_F32)
    out = []
    carry = jnp.zeros((1, o.shape[1]), _F32)
    for b in range(n // cb):
        ob = o[b * cb:(b + 1) * cb]
        out.append(jax.lax.dot_general(slt, ob, (((1,), (0,)), ((), ())),
                                       preferred_element_type=_F32,
                                       precision=_DEF) + carry)
        carry = carry + jnp.sum(ob, axis=0, keepdims=True)
    return jnp.concatenate(out, axis=0)


def _router_body(x_ref, gwt_ref, w_ref, pos_ref, te_ref, tv_ref,
                 fi_ref, pi_ref):
    x = x_ref[...]
    logits = jax.lax.dot_general(
        x, gwt_ref[...], (((1,), (0,)), ((), ())),
        preferred_element_type=_F32, precision=_DEF)
    m = jnp.max(logits, axis=-1, keepdims=True)
    p = jnp.exp(logits - m)
    scores = p / jnp.sum(p, axis=-1, keepdims=True)

    lane = lax.broadcasted_iota(jnp.int32, (T, E), 1)
    s1 = jnp.max(scores, axis=-1, keepdims=True)
    i1 = jnp.min(jnp.where(scores == s1, lane, E), axis=-1, keepdims=True)
    mask1 = lane == i1
    rest = jnp.where(mask1, -jnp.inf, scores)
    s2 = jnp.max(rest, axis=-1, keepdims=True)
    i2 = jnp.min(jnp.where(rest == s2, lane, E), axis=-1, keepdims=True)
    mask2 = lane == i2

    o1 = mask1.astype(_F32)
    o2 = mask2.astype(_F32)
    c1 = jnp.sum(o1, axis=0, keepdims=True)
    counts = c1 + jnp.sum(o2, axis=0, keepdims=True)

    fi_ref[...] = counts * (float(E) / float(A))
    pi_ref[...] = jnp.mean(scores, axis=0, keepdims=True)
    w_ref[...] = jnp.concatenate([s1, s2], axis=1)

    ptiles = jnp.floor((counts + (TILE - 1)) * (1.0 / TILE))
    ei = lax.broadcasted_iota(jnp.int32, (E, E), 0)
    ej = lax.broadcasted_iota(jnp.int32, (E, E), 1)
    sltE = (ei < ej).astype(_F32)
    tile_start = jax.lax.dot_general(ptiles, sltE, (((1,), (0,)), ((), ())),
                                     preferred_element_type=_F32,
                                     precision=_DEF)
    alignoff = tile_start * float(TILE)
    total_tiles = jnp.sum(ptiles, axis=1, keepdims=True)

    rank1 = _excl_cumsum_rows(o1)
    rank2 = _excl_cumsum_rows(o2) + c1
    pos1 = jnp.sum(o1 * (alignoff + rank1), axis=1, keepdims=True)
    pos2 = jnp.sum(o2 * (alignoff + rank2), axis=1, keepdims=True)
    pos_ref[...] = jnp.concatenate([pos1, pos2], axis=1).astype(jnp.int32)

    ti = lax.broadcasted_iota(jnp.int32, (NT, E), 0).astype(_F32)
    ts_b = jnp.broadcast_to(tile_start, (NT, E))
    te = jnp.sum((ti >= ts_b).astype(_F32), axis=1, keepdims=True) - 1.0
    te_ref[...] = jnp.clip(te, 0.0, float(E - 1)).astype(jnp.int32)
    tvi = lax.broadcasted_iota(jnp.int32, (NT, 1), 0).astype(_F32)
    tv_ref[...] = (tvi < total_tiles).astype(jnp.int32)


_BF16 = jnp.bfloat16


def _precast_body(eg_ref, eu_ref, ed_ref, sg_ref, su_ref, sd_ref,
                  ego_ref, euo_ref, edo_ref, sgo_ref, suo_ref, sdo_ref):
    ego_ref[...] = eg_ref[...].astype(_BF16)
    euo_ref[...] = eu_ref[...].astype(_BF16)
    edo_ref[...] = ed_ref[...].astype(_BF16)
    sgo_ref[...] = sg_ref[...].astype(_BF16)
    suo_ref[...] = su_ref[...].astype(_BF16)
    sdo_ref[...] = sd_ref[...].astype(_BF16)


def _grouped_body(te_ref, tv_ref, xg_ref, gw_ref, uw_ref, dw_ref, out_ref):
    i = pl.program_id(0)

    @pl.when(tv_ref[i] == 1)
    def _():
        xg = xg_ref[...].astype(_BF16)
        dims = (((1,), (1,)), ((), ()))
        g = jax.lax.dot_general(xg, gw_ref[0], dims,
                                preferred_element_type=_F32, precision=_DEF)
        u = jax.lax.dot_general(xg, uw_ref[0], dims,
                                preferred_element_type=_F32, precision=_DEF)
        act = (g * jax.nn.sigmoid(g) * u).astype(_BF16)
        out_ref[...] = jax.lax.dot_general(act, dw_ref[0], dims,
                                           preferred_element_type=_F32,
                                           precision=_DEF)


def _shared_body(x_ref, sg_ref, su_ref, sd_ref, y_ref):
    x = x_ref[...].astype(_BF16)
    dims = (((1,), (1,)), ((), ()))
    g = jax.lax.dot_general(x, sg_ref[...], dims,
                            preferred_element_type=_F32, precision=_DEF)
    u = jax.lax.dot_general(x, su_ref[...], dims,
                            preferred_element_type=_F32, precision=_DEF)
    act = (g * jax.nn.sigmoid(g) * u).astype(_BF16)
    y_ref[...] = jax.lax.dot_general(act, sd_ref[...], dims,
                                     preferred_element_type=_F32,
                                     precision=_DEF)


def _combine_body(eg1_ref, eg2_ref, w_ref, ysh_ref, y_ref):
    w1 = w_ref[:, 0]
    w2 = w_ref[:, 1]
    y_ref[...] = (w1[:, None] * eg1_ref[0] + w2[:, None] * eg2_ref[0]
                  + ysh_ref[...])


def _sc_mesh():
    return plsc.VectorSubcoreMesh(core_axis_name="c", subcore_axis_name="s")


_NW = 32
_JPW = A // _NW
_CH = 32
_NCH = _JPW // _CH


def _sc_scatter(x, pos3):
    @functools.partial(
        pl.kernel, mesh=_sc_mesh(),
        out_type=jax.ShapeDtypeStruct((NPAD, D), _F32),
        scratch_types=[pltpu.VMEM((_NCH, _CH), jnp.int32),
                       pltpu.VMEM((_CH, D), _F32),
                       pltpu.VMEM((_CH, D), _F32),
                       pltpu.SemaphoreType.DMA,
                       pltpu.SemaphoreType.DMA,
                       pltpu.SemaphoreType.DMA,
                       pltpu.SemaphoreType.DMA],
    )
    def k(x_hbm, pos_hbm, xg_hbm, idx_v, row0, row1, ld0, ld1, st0, st1):
        wid = lax.axis_index("s") * 2 + lax.axis_index("c")
        base = wid * _JPW
        pltpu.sync_copy(pos_hbm.at[wid], idx_v)
        rows = (row0, row1)
        lds = (ld0, ld1)
        sts = (st0, st1)

        def start_load(c, b):
            pltpu.async_copy(x_hbm.at[pl.ds((base + c * _CH) % T, _CH)],
                             rows[b], lds[b])

        start_load(0, 0)
        start_load(1, 1)

        @pl.loop(0, _NCH, step=2)
        def _(c):
            for b in range(2):
                cc = c + b
                pltpu.make_async_copy(
                    x_hbm.at[pl.ds(0, _CH)], rows[b], lds[b]).wait()
                cp = pltpu.async_copy(rows[b], xg_hbm.at[idx_v.at[cc]],
                                      sts[b])
                cp.wait()

                @pl.when(cc + 2 < _NCH)
                def _():
                    pltpu.async_copy(
                        x_hbm.at[pl.ds((base + (cc + 2) * _CH) % T, _CH)],
                        rows[b], lds[b])

    return k(x, pos3)


def _sc_gather(eout, pos3):
    @functools.partial(
        pl.kernel, mesh=_sc_mesh(),
        out_type=jax.ShapeDtypeStruct((A, D), _F32),
        scratch_types=[pltpu.VMEM((_NCH, _CH), jnp.int32),
                       pltpu.VMEM((_CH, D), _F32),
                       pltpu.VMEM((_CH, D), _F32),
                       pltpu.SemaphoreType.DMA,
                       pltpu.SemaphoreType.DMA,
                       pltpu.SemaphoreType.DMA,
                       pltpu.SemaphoreType.DMA],
    )
    def k(eout_hbm, pos_hbm, eg_hbm, idx_v, row0, row1, ld0, ld1, st0, st1):
        wid = lax.axis_index("s") * 2 + lax.axis_index("c")
        base = wid * _JPW
        pltpu.sync_copy(pos_hbm.at[wid], idx_v)
        rows = (row0, row1)
        lds = (ld0, ld1)
        sts = (st0, st1)

        def start_gather(c, b):
            pltpu.async_copy(eout_hbm.at[idx_v.at[c]], rows[b], lds[b])

        start_gather(0, 0)
        start_gather(1, 1)

        @pl.loop(0, _NCH, step=2)
        def _(c):
            for b in range(2):
                cc = c + b
                pltpu.make_async_copy(
                    eout_hbm.at[pl.ds(0, _CH)], rows[b], lds[b]).wait()
                cp = pltpu.async_copy(
                    rows[b], eg_hbm.at[pl.ds(base + cc * _CH, _CH)], sts[b])
                cp.wait()

                @pl.when(cc + 2 < _NCH)
                def _():
                    start_gather(cc + 2, b)

    return k(eout, pos3)


def kernel(hidden_states, gate_w, expert_gate, expert_up, expert_down,
           shared_gate, shared_up, shared_down):
    b, s, h = hidden_states.shape
    x = hidden_states.reshape(-1, h)

    w2, pos_tk, te, tv, fi, pi = pl.pallas_call(
        _router_body,
        out_shape=[
            jax.ShapeDtypeStruct((T, TOP_K), _F32),
            jax.ShapeDtypeStruct((T, TOP_K), jnp.int32),
            jax.ShapeDtypeStruct((NT, 1), jnp.int32),
            jax.ShapeDtypeStruct((NT, 1), jnp.int32),
            jax.ShapeDtypeStruct((1, E), _F32),
            jax.ShapeDtypeStruct((1, E), _F32),
        ],
    )(x, gate_w.T)

    pos3 = pos_tk.T.reshape(_NW, _NCH, _CH)
    te_s = te.reshape(NT)
    tv_s = tv.reshape(NT)

    half = DFF // 2
    eg_bf, eu_bf, ed_bf, sg_bf, su_bf, sd_bf = pl.pallas_call(
        _precast_body,
        grid=(16,),
        in_specs=[
            pl.BlockSpec((1, half, D), lambda i: (i // 2, i % 2, 0)),
            pl.BlockSpec((1, half, D), lambda i: (i // 2, i % 2, 0)),
            pl.BlockSpec((1, half, D), lambda i: (i // 2, i % 2, 0)),
            pl.BlockSpec((SH_DFF // 16, D), lambda i: (i, 0)),
            pl.BlockSpec((SH_DFF // 16, D), lambda i: (i, 0)),
            pl.BlockSpec((D // 16, SH_DFF), lambda i: (i, 0)),
        ],
        out_specs=[
            pl.BlockSpec((1, half, D), lambda i: (i // 2, i % 2, 0)),
            pl.BlockSpec((1, half, D), lambda i: (i // 2, i % 2, 0)),
            pl.BlockSpec((1, half, D), lambda i: (i // 2, i % 2, 0)),
            pl.BlockSpec((SH_DFF // 16, D), lambda i: (i, 0)),
            pl.BlockSpec((SH_DFF // 16, D), lambda i: (i, 0)),
            pl.BlockSpec((D // 16, SH_DFF), lambda i: (i, 0)),
        ],
        out_shape=[
            jax.ShapeDtypeStruct((E, DFF, D), _BF16),
            jax.ShapeDtypeStruct((E, DFF, D), _BF16),
            jax.ShapeDtypeStruct((E, D, DFF), _BF16),
            jax.ShapeDtypeStruct((SH_DFF, D), _BF16),
            jax.ShapeDtypeStruct((SH_DFF, D), _BF16),
            jax.ShapeDtypeStruct((D, SH_DFF), _BF16),
        ],
        compiler_params=pltpu.CompilerParams(
            dimension_semantics=("parallel",)),
    )(expert_gate, expert_up, expert_down, shared_gate, shared_up,
      shared_down)

    y_sh = pl.pallas_call(
        _shared_body,
        grid=(T // BT,),
        in_specs=[
            pl.BlockSpec((BT, D), lambda t: (t, 0)),
            pl.BlockSpec((SH_DFF, D), lambda t: (0, 0)),
            pl.BlockSpec((SH_DFF, D), lambda t: (0, 0)),
            pl.BlockSpec((D, SH_DFF), lambda t: (0, 0)),
        ],
        out_specs=pl.BlockSpec((BT, D), lambda t: (t, 0)),
        out_shape=jax.ShapeDtypeStruct((T, D), _F32),
        compiler_params=pltpu.CompilerParams(
            dimension_semantics=("parallel",)),
    )(x, sg_bf, su_bf, sd_bf)

    xg = _sc_scatter(x, pos3)

    eout = pl.pallas_call(
        _grouped_body,
        grid_spec=pltpu.PrefetchScalarGridSpec(
            num_scalar_prefetch=2,
            grid=(NT,),
            in_specs=[
                pl.BlockSpec((TILE, D), lambda i, te, tv: (i, 0)),
                pl.BlockSpec((1, DFF, D), lambda i, te, tv: (te[i], 0, 0)),
                pl.BlockSpec((1, DFF, D), lambda i, te, tv: (te[i], 0, 0)),
                pl.BlockSpec((1, D, DFF), lambda i, te, tv: (te[i], 0, 0)),
            ],
            out_specs=pl.BlockSpec((TILE, D), lambda i, te, tv: (i, 0)),
        ),
        out_shape=jax.ShapeDtypeStruct((NPAD, D), _F32),
        compiler_params=pltpu.CompilerParams(
            dimension_semantics=("arbitrary",)),
    )(te_s, tv_s, xg, eg_bf, eu_bf, ed_bf)

    eg = _sc_gather(eout, pos3).reshape(TOP_K, T, D)

    y = pl.pallas_call(
        _combine_body,
        grid=(T // BT,),
        in_specs=[
            pl.BlockSpec((1, BT, D), lambda t: (0, t, 0)),
            pl.BlockSpec((1, BT, D), lambda t: (1, t, 0)),
            pl.BlockSpec((BT, TOP_K), lambda t: (t, 0)),
            pl.BlockSpec((BT, D), lambda t: (t, 0)),
        ],
        out_specs=pl.BlockSpec((BT, D), lambda t: (t, 0)),
        out_shape=jax.ShapeDtypeStruct((T, D), _F32),
        compiler_params=pltpu.CompilerParams(
            dimension_semantics=("parallel",)),
    )(eg, eg, w2, y_sh)

    return (y.reshape(b, s, h), fi.reshape(E), pi.reshape(E))


# speedup vs baseline: 1.1129x; 1.1129x over previous
import functools

import jax
import jax.numpy as jnp
from jax import lax
from jax.experimental import pallas as pl
from jax.experimental.pallas import tpu as pltpu
from jax.experimental.pallas import tpu_sc as plsc

E = 8
TOP_K = 2
D = 1024
DFF = 1024
SH_DFF = 2048
T = 2048
A = T * TOP_K

TILE = 256
NT = A // TILE + E
NPAD = NT * TILE

BT = 256

_DEF = jax.lax.Precision.DEFAULT
_F32 = jnp.float32


def _excl_cumsum_rows(o, cb=256):
    n = o.shape[0]
    ii = lax.broadcasted_iota(jnp.int32, (cb, cb), 0)
    jj = lax.broadcasted_iota(jnp.int32, (cb, cb), 1)
    slt = (jj < ii).astype(_F32)
    out = []
    carry = jnp.zeros((1, o.shape[1]), _F32)
    for b in range(n // cb):
        ob = o[b * cb:(b + 1) * cb]
        out.append(jax.lax.dot_general(slt, ob, (((1,), (0,)), ((), ())),
                                       preferred_element_type=_F32,
                                       precision=_DEF) + carry)
        carry = carry + jnp.sum(ob, axis=0, keepdims=True)
    return jnp.concatenate(out, axis=0)


def _router_body(x_ref, gwt_ref, w_ref, pos_ref, te_ref, tv_ref,
                 fi_ref, pi_ref):
    x = x_ref[...]
    logits = jax.lax.dot_general(
        x, gwt_ref[...], (((1,), (0,)), ((), ())),
        preferred_element_type=_F32, precision=_DEF)
    m = jnp.max(logits, axis=-1, keepdims=True)
    p = jnp.exp(logits - m)
    scores = p / jnp.sum(p, axis=-1, keepdims=True)

    lane = lax.broadcasted_iota(jnp.int32, (T, E), 1)
    s1 = jnp.max(scores, axis=-1, keepdims=True)
    i1 = jnp.min(jnp.where(scores == s1, lane, E), axis=-1, keepdims=True)
    mask1 = lane == i1
    rest = jnp.where(mask1, -jnp.inf, scores)
    s2 = jnp.max(rest, axis=-1, keepdims=True)
    i2 = jnp.min(jnp.where(rest == s2, lane, E), axis=-1, keepdims=True)
    mask2 = lane == i2

    o1 = mask1.astype(_F32)
    o2 = mask2.astype(_F32)
    c1 = jnp.sum(o1, axis=0, keepdims=True)
    counts = c1 + jnp.sum(o2, axis=0, keepdims=True)

    fi_ref[...] = counts * (float(E) / float(A))
    pi_ref[...] = jnp.mean(scores, axis=0, keepdims=True)
    w_ref[...] = jnp.concatenate([s1, s2], axis=1)

    ptiles = jnp.floor((counts + (TILE - 1)) * (1.0 / TILE))
    ei = lax.broadcasted_iota(jnp.int32, (E, E), 0)
    ej = lax.broadcasted_iota(jnp.int32, (E, E), 1)
    sltE = (ei < ej).astype(_F32)
    tile_start = jax.lax.dot_general(ptiles, sltE, (((1,), (0,)), ((), ())),
                                     preferred_element_type=_F32,
                                     precision=_DEF)
    alignoff = tile_start * float(TILE)
    total_tiles = jnp.sum(ptiles, axis=1, keepdims=True)

    rank1 = _excl_cumsum_rows(o1)
    rank2 = _excl_cumsum_rows(o2) + c1
    pos1 = jnp.sum(o1 * (alignoff + rank1), axis=1, keepdims=True)
    pos2 = jnp.sum(o2 * (alignoff + rank2), axis=1, keepdims=True)
    pos_ref[...] = jnp.concatenate([pos1, pos2], axis=1).astype(jnp.int32)

    ti = lax.broadcasted_iota(jnp.int32, (NT, E), 0).astype(_F32)
    ts_b = jnp.broadcast_to(tile_start, (NT, E))
    te = jnp.sum((ti >= ts_b).astype(_F32), axis=1, keepdims=True) - 1.0
    te_ref[...] = jnp.clip(te, 0.0, float(E - 1)).astype(jnp.int32)
    tvi = lax.broadcasted_iota(jnp.int32, (NT, 1), 0).astype(_F32)
    tv_ref[...] = (tvi < total_tiles).astype(jnp.int32)


_BF16 = jnp.bfloat16


def _precast_body(eg_ref, eu_ref, ed_ref, sg_ref, su_ref, sd_ref,
                  ego_ref, euo_ref, edo_ref, sgo_ref, suo_ref, sdo_ref):
    ego_ref[...] = eg_ref[...].astype(_BF16)
    euo_ref[...] = eu_ref[...].astype(_BF16)
    edo_ref[...] = ed_ref[...].astype(_BF16)
    sgo_ref[...] = sg_ref[...].astype(_BF16)
    suo_ref[...] = su_ref[...].astype(_BF16)
    sdo_ref[...] = sd_ref[...].astype(_BF16)


def _grouped_body(te_ref, tv_ref, xg_ref, gw_ref, uw_ref, dw_ref, out_ref):
    i = pl.program_id(0)

    @pl.when(tv_ref[i] == 1)
    def _():
        xg = xg_ref[...]
        dims = (((1,), (1,)), ((), ()))
        g = jax.lax.dot_general(xg, gw_ref[0], dims,
                                preferred_element_type=_F32, precision=_DEF)
        u = jax.lax.dot_general(xg, uw_ref[0], dims,
                                preferred_element_type=_F32, precision=_DEF)
        act = g * jax.nn.sigmoid(g) * u
        out_ref[...] = jax.lax.dot_general(act, dw_ref[0], dims,
                                           preferred_element_type=_F32,
                                           precision=_DEF)


def _shared_body(x_ref, sg_ref, su_ref, sd_ref, y_ref):
    x = x_ref[...]
    dims = (((1,), (1,)), ((), ()))
    g = jax.lax.dot_general(x, sg_ref[...], dims,
                            preferred_element_type=_F32, precision=_DEF)
    u = jax.lax.dot_general(x, su_ref[...], dims,
                            preferred_element_type=_F32, precision=_DEF)
    act = g * jax.nn.sigmoid(g) * u
    y_ref[...] = jax.lax.dot_general(act, sd_ref[...], dims,
                                     preferred_element_type=_F32,
                                     precision=_DEF)


def _combine_body(eg1_ref, eg2_ref, w_ref, ysha_ref, yshb_ref, y_ref):
    t = pl.program_id(0)
    w1 = w_ref[:, 0]
    w2 = w_ref[:, 1]
    nh = T // 2 // BT
    ysh = jnp.where(t < nh, ysha_ref[...], yshb_ref[...])
    y_ref[...] = (w1[:, None] * eg1_ref[0] + w2[:, None] * eg2_ref[0]
                  + ysh)


def _sc_mesh():
    return plsc.VectorSubcoreMesh(core_axis_name="c", subcore_axis_name="s")


_NW = 32
_JPW = A // _NW
_CH = 32
_NCH = _JPW // _CH


def _sc_scatter(x, pos3):
    @functools.partial(
        pl.kernel, mesh=_sc_mesh(),
        out_type=jax.ShapeDtypeStruct((NPAD, D), _F32),
        scratch_types=[pltpu.VMEM((_NCH, _CH), jnp.int32),
                       pltpu.VMEM((_CH, D), _F32),
                       pltpu.VMEM((_CH, D), _F32),
                       pltpu.SemaphoreType.DMA,
                       pltpu.SemaphoreType.DMA,
                       pltpu.SemaphoreType.DMA,
                       pltpu.SemaphoreType.DMA],
    )
    def k(x_hbm, pos_hbm, xg_hbm, idx_v, row0, row1, ld0, ld1, st0, st1):
        wid = lax.axis_index("s") * 2 + lax.axis_index("c")
        base = wid * _JPW
        pltpu.sync_copy(pos_hbm.at[wid], idx_v)
        rows = (row0, row1)
        lds = (ld0, ld1)
        sts = (st0, st1)

        def start_load(c, b):
            pltpu.async_copy(x_hbm.at[pl.ds((base + c * _CH) % T, _CH)],
                             rows[b], lds[b])

        start_load(0, 0)
        start_load(1, 1)

        @pl.loop(0, _NCH, step=2)
        def _(c):
            for b in range(2):
                cc = c + b
                pltpu.make_async_copy(
                    x_hbm.at[pl.ds(0, _CH)], rows[b], lds[b]).wait()
                cp = pltpu.async_copy(rows[b], xg_hbm.at[idx_v.at[cc]],
                                      sts[b])
                cp.wait()

                @pl.when(cc + 2 < _NCH)
                def _():
                    pltpu.async_copy(
                        x_hbm.at[pl.ds((base + (cc + 2) * _CH) % T, _CH)],
                        rows[b], lds[b])

    return k(x, pos3)


def _sc_gather(eout, pos3):
    @functools.partial(
        pl.kernel, mesh=_sc_mesh(),
        out_type=jax.ShapeDtypeStruct((A, D), _F32),
        scratch_types=[pltpu.VMEM((_NCH, _CH), jnp.int32),
                       pltpu.VMEM((_CH, D), _F32),
                       pltpu.VMEM((_CH, D), _F32),
                       pltpu.SemaphoreType.DMA,
                       pltpu.SemaphoreType.DMA,
                       pltpu.SemaphoreType.DMA,
                       pltpu.SemaphoreType.DMA],
    )
    def k(eout_hbm, pos_hbm, eg_hbm, idx_v, row0, row1, ld0, ld1, st0, st1):
        wid = lax.axis_index("s") * 2 + lax.axis_index("c")
        base = wid * _JPW
        pltpu.sync_copy(pos_hbm.at[wid], idx_v)
        rows = (row0, row1)
        lds = (ld0, ld1)
        sts = (st0, st1)

        def start_gather(c, b):
            pltpu.async_copy(eout_hbm.at[idx_v.at[c]], rows[b], lds[b])

        start_gather(0, 0)
        start_gather(1, 1)

        @pl.loop(0, _NCH, step=2)
        def _(c):
            for b in range(2):
                cc = c + b
                pltpu.make_async_copy(
                    eout_hbm.at[pl.ds(0, _CH)], rows[b], lds[b]).wait()
                cp = pltpu.async_copy(
                    rows[b], eg_hbm.at[pl.ds(base + cc * _CH, _CH)], sts[b])
                cp.wait()

                @pl.when(cc + 2 < _NCH)
                def _():
                    start_gather(cc + 2, b)

    return k(eout, pos3)


def kernel(hidden_states, gate_w, expert_gate, expert_up, expert_down,
           shared_gate, shared_up, shared_down):
    b, s, h = hidden_states.shape
    x = hidden_states.reshape(-1, h)

    w2, pos_tk, te, tv, fi, pi = pl.pallas_call(
        _router_body,
        out_shape=[
            jax.ShapeDtypeStruct((T, TOP_K), _F32),
            jax.ShapeDtypeStruct((T, TOP_K), jnp.int32),
            jax.ShapeDtypeStruct((NT, 1), jnp.int32),
            jax.ShapeDtypeStruct((NT, 1), jnp.int32),
            jax.ShapeDtypeStruct((1, E), _F32),
            jax.ShapeDtypeStruct((1, E), _F32),
        ],
    )(x, gate_w.T)

    pos3 = pos_tk.T.reshape(_NW, _NCH, _CH)
    te_s = te.reshape(NT)
    tv_s = tv.reshape(NT)


    def _shared_half(off):
        return pl.pallas_call(
            _shared_body,
            grid=(T // 2 // BT,),
            in_specs=[
                pl.BlockSpec((BT, D), lambda t: (t + off, 0)),
                pl.BlockSpec((SH_DFF, D), lambda t: (0, 0)),
                pl.BlockSpec((SH_DFF, D), lambda t: (0, 0)),
                pl.BlockSpec((D, SH_DFF), lambda t: (0, 0)),
            ],
            out_specs=pl.BlockSpec((BT, D), lambda t: (t, 0)),
            out_shape=jax.ShapeDtypeStruct((T // 2, D), _F32),
            compiler_params=pltpu.CompilerParams(
                dimension_semantics=("arbitrary",)),
        )(x, shared_gate, shared_up, shared_down)

    y_sh_a = _shared_half(0)
    xg = _sc_scatter(x, pos3)
    y_sh_b = _shared_half(T // 2 // BT)

    eout = pl.pallas_call(
        _grouped_body,
        grid_spec=pltpu.PrefetchScalarGridSpec(
            num_scalar_prefetch=2,
            grid=(NT,),
            in_specs=[
                pl.BlockSpec((TILE, D), lambda i, te, tv: (i, 0)),
                pl.BlockSpec((1, DFF, D), lambda i, te, tv: (te[i], 0, 0)),
                pl.BlockSpec((1, DFF, D), lambda i, te, tv: (te[i], 0, 0)),
                pl.BlockSpec((1, D, DFF), lambda i, te, tv: (te[i], 0, 0)),
            ],
            out_specs=pl.BlockSpec((TILE, D), lambda i, te, tv: (i, 0)),
        ),
        out_shape=jax.ShapeDtypeStruct((NPAD, D), _F32),
        compiler_params=pltpu.CompilerParams(
            dimension_semantics=("arbitrary",)),
    )(te_s, tv_s, xg, expert_gate, expert_up, expert_down)

    eg = _sc_gather(eout, pos3).reshape(TOP_K, T, D)

    y = pl.pallas_call(
        _combine_body,
        grid=(T // BT,),
        in_specs=[
            pl.BlockSpec((1, BT, D), lambda t: (0, t, 0)),
            pl.BlockSpec((1, BT, D), lambda t: (1, t, 0)),
            pl.BlockSpec((BT, TOP_K), lambda t: (t, 0)),
            pl.BlockSpec((BT, D),
                         lambda t: (jnp.minimum(t, T // 2 // BT - 1), 0)),
            pl.BlockSpec((BT, D),
                         lambda t: (jnp.maximum(t - T // 2 // BT, 0), 0)),
        ],
        out_specs=pl.BlockSpec((BT, D), lambda t: (t, 0)),
        out_shape=jax.ShapeDtypeStruct((T, D), _F32),
        compiler_params=pltpu.CompilerParams(
            dimension_semantics=("parallel",)),
    )(eg, eg, w2, y_sh_a, y_sh_b)

    return (y.reshape(b, s, h), fi.reshape(E), pi.reshape(E))


# speedup vs baseline: 1.1174x; 1.0041x over previous
import functools

import jax
import jax.numpy as jnp
from jax import lax
from jax.experimental import pallas as pl
from jax.experimental.pallas import tpu as pltpu
from jax.experimental.pallas import tpu_sc as plsc

E = 8
TOP_K = 2
D = 1024
DFF = 1024
SH_DFF = 2048
T = 2048
A = T * TOP_K

TILE = 256
NT = A // TILE + E
NPAD = NT * TILE

BT = 256

_DEF = jax.lax.Precision.DEFAULT
_F32 = jnp.float32


def _excl_cumsum_rows(o, cb=256):
    n = o.shape[0]
    ii = lax.broadcasted_iota(jnp.int32, (cb, cb), 0)
    jj = lax.broadcasted_iota(jnp.int32, (cb, cb), 1)
    slt = (jj < ii).astype(_F32)
    out = []
    carry = jnp.zeros((1, o.shape[1]), _F32)
    for b in range(n // cb):
        ob = o[b * cb:(b + 1) * cb]
        out.append(jax.lax.dot_general(slt, ob, (((1,), (0,)), ((), ())),
                                       preferred_element_type=_F32,
                                       precision=_DEF) + carry)
        carry = carry + jnp.sum(ob, axis=0, keepdims=True)
    return jnp.concatenate(out, axis=0)


def _router_body(x_ref, gwt_ref, w_ref, pos_ref, te_ref, tv_ref,
                 fi_ref, pi_ref):
    x = x_ref[...]
    logits = jax.lax.dot_general(
        x, gwt_ref[...], (((1,), (0,)), ((), ())),
        preferred_element_type=_F32, precision=_DEF)
    m = jnp.max(logits, axis=-1, keepdims=True)
    p = jnp.exp(logits - m)
    scores = p / jnp.sum(p, axis=-1, keepdims=True)

    lane = lax.broadcasted_iota(jnp.int32, (T, E), 1)
    s1 = jnp.max(scores, axis=-1, keepdims=True)
    i1 = jnp.min(jnp.where(scores == s1, lane, E), axis=-1, keepdims=True)
    mask1 = lane == i1
    rest = jnp.where(mask1, -jnp.inf, scores)
    s2 = jnp.max(rest, axis=-1, keepdims=True)
    i2 = jnp.min(jnp.where(rest == s2, lane, E), axis=-1, keepdims=True)
    mask2 = lane == i2

    o1 = mask1.astype(_F32)
    o2 = mask2.astype(_F32)
    c1 = jnp.sum(o1, axis=0, keepdims=True)
    counts = c1 + jnp.sum(o2, axis=0, keepdims=True)

    fi_ref[...] = counts * (float(E) / float(A))
    pi_ref[...] = jnp.mean(scores, axis=0, keepdims=True)
    w_ref[...] = jnp.concatenate([s1, s2], axis=1)

    ptiles = jnp.floor((counts + (TILE - 1)) * (1.0 / TILE))
    ei = lax.broadcasted_iota(jnp.int32, (E, E), 0)
    ej = lax.broadcasted_iota(jnp.int32, (E, E), 1)
    sltE = (ei < ej).astype(_F32)
    tile_start = jax.lax.dot_general(ptiles, sltE, (((1,), (0,)), ((), ())),
                                     preferred_element_type=_F32,
                                     precision=_DEF)
    alignoff = tile_start * float(TILE)
    total_tiles = jnp.sum(ptiles, axis=1, keepdims=True)

    rank1 = _excl_cumsum_rows(o1)
    rank2 = _excl_cumsum_rows(o2) + c1
    pos1 = jnp.sum(o1 * (alignoff + rank1), axis=1, keepdims=True)
    pos2 = jnp.sum(o2 * (alignoff + rank2), axis=1, keepdims=True)
    pos_ref[...] = jnp.concatenate([pos1, pos2], axis=1).astype(jnp.int32)

    ti = lax.broadcasted_iota(jnp.int32, (NT, E), 0).astype(_F32)
    ts_b = jnp.broadcast_to(tile_start, (NT, E))
    te = jnp.sum((ti >= ts_b).astype(_F32), axis=1, keepdims=True) - 1.0
    te_ref[...] = jnp.clip(te, 0.0, float(E - 1)).astype(jnp.int32)
    tvi = lax.broadcasted_iota(jnp.int32, (NT, 1), 0).astype(_F32)
    tv_ref[...] = (tvi < total_tiles).astype(jnp.int32)


_BF16 = jnp.bfloat16


def _precast_body(eg_ref, eu_ref, ed_ref, sg_ref, su_ref, sd_ref,
                  ego_ref, euo_ref, edo_ref, sgo_ref, suo_ref, sdo_ref):
    ego_ref[...] = eg_ref[...].astype(_BF16)
    euo_ref[...] = eu_ref[...].astype(_BF16)
    edo_ref[...] = ed_ref[...].astype(_BF16)
    sgo_ref[...] = sg_ref[...].astype(_BF16)
    suo_ref[...] = su_ref[...].astype(_BF16)
    sdo_ref[...] = sd_ref[...].astype(_BF16)


def _grouped_body(te_ref, tv_ref, xg_ref, gw_ref, uw_ref, dw_ref, dep_ref,
                  out_ref):
    del dep_ref
    i = pl.program_id(0)

    @pl.when(tv_ref[i] == 1)
    def _():
        xg = xg_ref[...]
        dims = (((1,), (1,)), ((), ()))
        g = jax.lax.dot_general(xg, gw_ref[0], dims,
                                preferred_element_type=_F32, precision=_DEF)
        u = jax.lax.dot_general(xg, uw_ref[0], dims,
                                preferred_element_type=_F32, precision=_DEF)
        act = g * jax.nn.sigmoid(g) * u
        out_ref[...] = jax.lax.dot_general(act, dw_ref[0], dims,
                                           preferred_element_type=_F32,
                                           precision=_DEF)


def _shared_body(x_ref, sg_ref, su_ref, sd_ref, y_ref):
    x = x_ref[...]
    dims = (((1,), (1,)), ((), ()))
    g = jax.lax.dot_general(x, sg_ref[...], dims,
                            preferred_element_type=_F32, precision=_DEF)
    u = jax.lax.dot_general(x, su_ref[...], dims,
                            preferred_element_type=_F32, precision=_DEF)
    act = g * jax.nn.sigmoid(g) * u
    y_ref[...] = jax.lax.dot_general(act, sd_ref[...], dims,
                                     preferred_element_type=_F32,
                                     precision=_DEF)


def _combine_body(eg1_ref, eg2_ref, w_ref, ysha_ref, yshb_ref, y_ref):
    t = pl.program_id(0)
    w1 = w_ref[:, 0]
    w2 = w_ref[:, 1]
    nh = T // 2 // BT
    ysh = jnp.where(t < nh, ysha_ref[...], yshb_ref[...])
    y_ref[...] = (w1[:, None] * eg1_ref[0] + w2[:, None] * eg2_ref[0]
                  + ysh)


def _sc_mesh():
    return plsc.VectorSubcoreMesh(core_axis_name="c", subcore_axis_name="s")


_NW = 32
_JPW = A // _NW
_CH = 32
_NCH = _JPW // _CH


def _sc_scatter(x, pos3):
    @functools.partial(
        pl.kernel, mesh=_sc_mesh(),
        out_type=jax.ShapeDtypeStruct((NPAD, D), _F32),
        scratch_types=[pltpu.VMEM((_NCH, _CH), jnp.int32),
                       pltpu.VMEM((_CH, D), _F32),
                       pltpu.VMEM((_CH, D), _F32),
                       pltpu.SemaphoreType.DMA,
                       pltpu.SemaphoreType.DMA,
                       pltpu.SemaphoreType.DMA,
                       pltpu.SemaphoreType.DMA],
    )
    def k(x_hbm, pos_hbm, xg_hbm, idx_v, row0, row1, ld0, ld1, st0, st1):
        wid = lax.axis_index("s") * 2 + lax.axis_index("c")
        base = wid * _JPW
        pltpu.sync_copy(pos_hbm.at[wid], idx_v)
        rows = (row0, row1)
        lds = (ld0, ld1)
        sts = (st0, st1)

        def start_load(c, b):
            pltpu.async_copy(x_hbm.at[pl.ds((base + c * _CH) % T, _CH)],
                             rows[b], lds[b])

        start_load(0, 0)
        start_load(1, 1)

        @pl.loop(0, _NCH, step=2)
        def _(c):
            for b in range(2):
                cc = c + b
                pltpu.make_async_copy(
                    x_hbm.at[pl.ds(0, _CH)], rows[b], lds[b]).wait()
                cp = pltpu.async_copy(rows[b], xg_hbm.at[idx_v.at[cc]],
                                      sts[b])
                cp.wait()

                @pl.when(cc + 2 < _NCH)
                def _():
                    pltpu.async_copy(
                        x_hbm.at[pl.ds((base + (cc + 2) * _CH) % T, _CH)],
                        rows[b], lds[b])

    return k(x, pos3)


def _sc_gather(eout, pos3):
    @functools.partial(
        pl.kernel, mesh=_sc_mesh(),
        out_type=jax.ShapeDtypeStruct((A, D), _F32),
        scratch_types=[pltpu.VMEM((_NCH, _CH), jnp.int32),
                       pltpu.VMEM((_CH, D), _F32),
                       pltpu.VMEM((_CH, D), _F32),
                       pltpu.SemaphoreType.DMA,
                       pltpu.SemaphoreType.DMA,
                       pltpu.SemaphoreType.DMA,
                       pltpu.SemaphoreType.DMA],
    )
    def k(eout_hbm, pos_hbm, eg_hbm, idx_v, row0, row1, ld0, ld1, st0, st1):
        wid = lax.axis_index("s") * 2 + lax.axis_index("c")
        base = wid * _JPW
        pltpu.sync_copy(pos_hbm.at[wid], idx_v)
        rows = (row0, row1)
        lds = (ld0, ld1)
        sts = (st0, st1)

        def start_gather(c, b):
            pltpu.async_copy(eout_hbm.at[idx_v.at[c]], rows[b], lds[b])

        start_gather(0, 0)
        start_gather(1, 1)

        @pl.loop(0, _NCH, step=2)
        def _(c):
            for b in range(2):
                cc = c + b
                pltpu.make_async_copy(
                    eout_hbm.at[pl.ds(0, _CH)], rows[b], lds[b]).wait()
                cp = pltpu.async_copy(
                    rows[b], eg_hbm.at[pl.ds(base + cc * _CH, _CH)], sts[b])
                cp.wait()

                @pl.when(cc + 2 < _NCH)
                def _():
                    start_gather(cc + 2, b)

    return k(eout, pos3)


def kernel(hidden_states, gate_w, expert_gate, expert_up, expert_down,
           shared_gate, shared_up, shared_down):
    b, s, h = hidden_states.shape
    x = hidden_states.reshape(-1, h)

    w2, pos_tk, te, tv, fi, pi = pl.pallas_call(
        _router_body,
        out_shape=[
            jax.ShapeDtypeStruct((T, TOP_K), _F32),
            jax.ShapeDtypeStruct((T, TOP_K), jnp.int32),
            jax.ShapeDtypeStruct((NT, 1), jnp.int32),
            jax.ShapeDtypeStruct((NT, 1), jnp.int32),
            jax.ShapeDtypeStruct((1, E), _F32),
            jax.ShapeDtypeStruct((1, E), _F32),
        ],
    )(x, gate_w.T)

    pos3 = pos_tk.T.reshape(_NW, _NCH, _CH)
    te_s = te.reshape(NT)
    tv_s = tv.reshape(NT)


    def _shared_half(off):
        return pl.pallas_call(
            _shared_body,
            grid=(T // 2 // BT,),
            in_specs=[
                pl.BlockSpec((BT, D), lambda t: (t + off, 0)),
                pl.BlockSpec((SH_DFF, D), lambda t: (0, 0)),
                pl.BlockSpec((SH_DFF, D), lambda t: (0, 0)),
                pl.BlockSpec((D, SH_DFF), lambda t: (0, 0)),
            ],
            out_specs=pl.BlockSpec((BT, D), lambda t: (t, 0)),
            out_shape=jax.ShapeDtypeStruct((T // 2, D), _F32),
            compiler_params=pltpu.CompilerParams(
                dimension_semantics=("arbitrary",)),
        )(x, shared_gate, shared_up, shared_down)

    y_sh_a = _shared_half(0)
    xg = _sc_scatter(x, pos3)
    y_sh_b = _shared_half(T // 2 // BT)

    eout = pl.pallas_call(
        _grouped_body,
        grid_spec=pltpu.PrefetchScalarGridSpec(
            num_scalar_prefetch=2,
            grid=(NT,),
            in_specs=[
                pl.BlockSpec((TILE, D), lambda i, te, tv: (i, 0)),
                pl.BlockSpec((1, DFF, D), lambda i, te, tv: (te[i], 0, 0)),
                pl.BlockSpec((1, DFF, D), lambda i, te, tv: (te[i], 0, 0)),
                pl.BlockSpec((1, D, DFF), lambda i, te, tv: (te[i], 0, 0)),
                pl.BlockSpec((8, 128), lambda i, te, tv: (0, 0)),
            ],
            out_specs=pl.BlockSpec((TILE, D), lambda i, te, tv: (i, 0)),
        ),
        out_shape=jax.ShapeDtypeStruct((NPAD, D), _F32),
        compiler_params=pltpu.CompilerParams(
            dimension_semantics=("arbitrary",)),
    )(te_s, tv_s, xg, expert_gate, expert_up, expert_down, y_sh_a)

    eg = _sc_gather(eout, pos3).reshape(TOP_K, T, D)

    y = pl.pallas_call(
        _combine_body,
        grid=(T // BT,),
        in_specs=[
            pl.BlockSpec((1, BT, D), lambda t: (0, t, 0)),
            pl.BlockSpec((1, BT, D), lambda t: (1, t, 0)),
            pl.BlockSpec((BT, TOP_K), lambda t: (t, 0)),
            pl.BlockSpec((BT, D),
                         lambda t: (jnp.minimum(t, T // 2 // BT - 1), 0)),
            pl.BlockSpec((BT, D),
                         lambda t: (jnp.maximum(t - T // 2 // BT, 0), 0)),
        ],
        out_specs=pl.BlockSpec((BT, D), lambda t: (t, 0)),
        out_shape=jax.ShapeDtypeStruct((T, D), _F32),
        compiler_params=pltpu.CompilerParams(
            dimension_semantics=("parallel",)),
    )(eg, eg, w2, y_sh_a, y_sh_b)

    return (y.reshape(b, s, h), fi.reshape(E), pi.reshape(E))


# speedup vs baseline: 1.2036x; 1.0771x over previous
import functools

import jax
import jax.numpy as jnp
from jax import lax
from jax.experimental import pallas as pl
from jax.experimental.pallas import tpu as pltpu
from jax.experimental.pallas import tpu_sc as plsc

E = 8
TOP_K = 2
D = 1024
DFF = 1024
SH_DFF = 2048
T = 2048
A = T * TOP_K

TILE = 256
NT = A // TILE + E
NPAD = NT * TILE

BT = 256

_DEF = jax.lax.Precision.DEFAULT
_F32 = jnp.float32


def _excl_cumsum_rows(o, cb=256):
    n = o.shape[0]
    ii = lax.broadcasted_iota(jnp.int32, (cb, cb), 0)
    jj = lax.broadcasted_iota(jnp.int32, (cb, cb), 1)
    slt = (jj < ii).astype(_F32)
    out = []
    carry = jnp.zeros((1, o.shape[1]), _F32)
    for b in range(n // cb):
        ob = o[b * cb:(b + 1) * cb]
        out.append(jax.lax.dot_general(slt, ob, (((1,), (0,)), ((), ())),
                                       preferred_element_type=_F32,
                                       precision=_DEF) + carry)
        carry = carry + jnp.sum(ob, axis=0, keepdims=True)
    return jnp.concatenate(out, axis=0)


def _router_body(x_ref, gwt_ref, w_ref, pos_ref, te_ref, tv_ref,
                 tn_ref, slot_ref, nxte_ref, pf_ref, fi_ref, pi_ref):
    x = x_ref[...]
    logits = jax.lax.dot_general(
        x, gwt_ref[...], (((1,), (0,)), ((), ())),
        preferred_element_type=_F32, precision=_DEF)
    m = jnp.max(logits, axis=-1, keepdims=True)
    p = jnp.exp(logits - m)
    scores = p / jnp.sum(p, axis=-1, keepdims=True)

    lane = lax.broadcasted_iota(jnp.int32, (T, E), 1)
    s1 = jnp.max(scores, axis=-1, keepdims=True)
    i1 = jnp.min(jnp.where(scores == s1, lane, E), axis=-1, keepdims=True)
    mask1 = lane == i1
    rest = jnp.where(mask1, -jnp.inf, scores)
    s2 = jnp.max(rest, axis=-1, keepdims=True)
    i2 = jnp.min(jnp.where(rest == s2, lane, E), axis=-1, keepdims=True)
    mask2 = lane == i2

    o1 = mask1.astype(_F32)
    o2 = mask2.astype(_F32)
    c1 = jnp.sum(o1, axis=0, keepdims=True)
    counts = c1 + jnp.sum(o2, axis=0, keepdims=True)

    fi_ref[...] = counts * (float(E) / float(A))
    pi_ref[...] = jnp.mean(scores, axis=0, keepdims=True)
    w_ref[...] = jnp.concatenate([s1, s2], axis=1)

    ptiles = jnp.floor((counts + (TILE - 1)) * (1.0 / TILE))
    ei = lax.broadcasted_iota(jnp.int32, (E, E), 0)
    ej = lax.broadcasted_iota(jnp.int32, (E, E), 1)
    sltE = (ei < ej).astype(_F32)
    tile_start = jax.lax.dot_general(ptiles, sltE, (((1,), (0,)), ((), ())),
                                     preferred_element_type=_F32,
                                     precision=_DEF)
    alignoff = tile_start * float(TILE)
    total_tiles = jnp.sum(ptiles, axis=1, keepdims=True)

    rank1 = _excl_cumsum_rows(o1)
    rank2 = _excl_cumsum_rows(o2) + c1
    pos1 = jnp.sum(o1 * (alignoff + rank1), axis=1, keepdims=True)
    pos2 = jnp.sum(o2 * (alignoff + rank2), axis=1, keepdims=True)
    pos_ref[...] = jnp.concatenate([pos1, pos2], axis=1).astype(jnp.int32)

    ti = lax.broadcasted_iota(jnp.int32, (NT, E), 0).astype(_F32)
    ts_b = jnp.broadcast_to(tile_start, (NT, E))
    te = jnp.sum((ti >= ts_b).astype(_F32), axis=1, keepdims=True) - 1.0
    te = jnp.clip(te, 0.0, float(E - 1))
    te_ref[...] = te.astype(jnp.int32)
    tvi = lax.broadcasted_iota(jnp.int32, (NT, 1), 0).astype(_F32)
    tv = (tvi < total_tiles).astype(_F32)
    tv_ref[...] = tv.astype(jnp.int32)

    te_prev = jnp.concatenate([-jnp.ones((1, 1), _F32), te[:-1]], axis=0)
    tn = (te != te_prev).astype(_F32) * tv
    ri = lax.broadcasted_iota(jnp.int32, (NT, NT), 0)
    rj = lax.broadcasted_iota(jnp.int32, (NT, NT), 1)
    incl = (rj <= ri).astype(_F32)
    run_idx = jax.lax.dot_general(incl, tn, (((1,), (0,)), ((), ())),
                                  preferred_element_type=_F32,
                                  precision=_DEF)
    m = run_idx - 1.0
    slot = m - 2.0 * jnp.floor(m * 0.5)
    slot_ref[...] = slot.astype(jnp.int32)
    nxs = jnp.min(jnp.where(ts_b > ti, ts_b, float(NT)), axis=1,
                  keepdims=True)
    hasnext = (nxs < total_tiles).astype(_F32)
    nxte = jnp.sum((ts_b <= jnp.broadcast_to(nxs, (NT, E))).astype(_F32),
                   axis=1, keepdims=True) - 1.0
    nxte_ref[...] = jnp.clip(nxte, 0.0, float(E - 1)).astype(jnp.int32)
    tn_ref[...] = tn.astype(jnp.int32)
    pf_ref[...] = (tn * hasnext).astype(jnp.int32)


_BF16 = jnp.bfloat16


def _precast_body(eg_ref, eu_ref, ed_ref, sg_ref, su_ref, sd_ref,
                  ego_ref, euo_ref, edo_ref, sgo_ref, suo_ref, sdo_ref):
    ego_ref[...] = eg_ref[...].astype(_BF16)
    euo_ref[...] = eu_ref[...].astype(_BF16)
    edo_ref[...] = ed_ref[...].astype(_BF16)
    sgo_ref[...] = sg_ref[...].astype(_BF16)
    suo_ref[...] = su_ref[...].astype(_BF16)
    sdo_ref[...] = sd_ref[...].astype(_BF16)


def _grouped_body(te_ref, tv_ref, tn_ref, slot_ref, nxte_ref, pf_ref,
                  xg_ref, gwh_ref, uwh_ref, dwh_ref, dep_ref, out_ref,
                  gw_v, uw_v, dw_v, gsem, usem, dsem):
    del dep_ref
    i = pl.program_id(0)
    slot = slot_ref[i]

    @pl.when(i == 0)
    def _():
        e = te_ref[0]
        c1 = pltpu.make_async_copy(gwh_ref.at[e], gw_v.at[0], gsem.at[0])
        c2 = pltpu.make_async_copy(uwh_ref.at[e], uw_v.at[0], usem.at[0])
        c3 = pltpu.make_async_copy(dwh_ref.at[e], dw_v.at[0], dsem.at[0])
        c1.start(); c2.start(); c3.start()
        c1.wait(); c2.wait(); c3.wait()

    @pl.when((i > 0) & (tn_ref[i] == 1) & (tv_ref[i] == 1))
    def _():
        e = te_ref[i]
        pltpu.make_async_copy(gwh_ref.at[e], gw_v.at[slot],
                              gsem.at[slot]).wait()
        pltpu.make_async_copy(uwh_ref.at[e], uw_v.at[slot],
                              usem.at[slot]).wait()
        pltpu.make_async_copy(dwh_ref.at[e], dw_v.at[slot],
                              dsem.at[slot]).wait()

    @pl.when(pf_ref[i] == 1)
    def _():
        ne = nxte_ref[i]
        other = 1 - slot
        pltpu.make_async_copy(gwh_ref.at[ne], gw_v.at[other],
                              gsem.at[other]).start()
        pltpu.make_async_copy(uwh_ref.at[ne], uw_v.at[other],
                              usem.at[other]).start()
        pltpu.make_async_copy(dwh_ref.at[ne], dw_v.at[other],
                              dsem.at[other]).start()

    @pl.when(tv_ref[i] == 1)
    def _():
        xg = xg_ref[...]
        dims = (((1,), (1,)), ((), ()))
        g = jax.lax.dot_general(xg, gw_v[slot], dims,
                                preferred_element_type=_F32, precision=_DEF)
        u = jax.lax.dot_general(xg, uw_v[slot], dims,
                                preferred_element_type=_F32, precision=_DEF)
        act = g * jax.nn.sigmoid(g) * u
        out_ref[...] = jax.lax.dot_general(act, dw_v[slot], dims,
                                           preferred_element_type=_F32,
                                           precision=_DEF)


def _shared_body(x_ref, sg_ref, su_ref, sd_ref, y_ref):
    x = x_ref[...]
    dims = (((1,), (1,)), ((), ()))
    g = jax.lax.dot_general(x, sg_ref[...], dims,
                            preferred_element_type=_F32, precision=_DEF)
    u = jax.lax.dot_general(x, su_ref[...], dims,
                            preferred_element_type=_F32, precision=_DEF)
    act = g * jax.nn.sigmoid(g) * u
    y_ref[...] = jax.lax.dot_general(act, sd_ref[...], dims,
                                     preferred_element_type=_F32,
                                     precision=_DEF)


def _combine_body(eg1_ref, eg2_ref, w_ref, ysha_ref, yshb_ref, y_ref):
    t = pl.program_id(0)
    w1 = w_ref[:, 0]
    w2 = w_ref[:, 1]
    nh = T // 2 // BT
    ysh = jnp.where(t < nh, ysha_ref[...], yshb_ref[...])
    y_ref[...] = (w1[:, None] * eg1_ref[0] + w2[:, None] * eg2_ref[0]
                  + ysh)


def _sc_mesh():
    return plsc.VectorSubcoreMesh(core_axis_name="c", subcore_axis_name="s")


_NW = 32
_JPW = A // _NW
_CH = 32
_NCH = _JPW // _CH


def _sc_scatter(x, pos3):
    @functools.partial(
        pl.kernel, mesh=_sc_mesh(),
        out_type=jax.ShapeDtypeStruct((NPAD, D), _F32),
        scratch_types=[pltpu.VMEM((_NCH, _CH), jnp.int32),
                       pltpu.VMEM((_CH, D), _F32),
                       pltpu.VMEM((_CH, D), _F32),
                       pltpu.SemaphoreType.DMA,
                       pltpu.SemaphoreType.DMA,
                       pltpu.SemaphoreType.DMA,
                       pltpu.SemaphoreType.DMA],
    )
    def k(x_hbm, pos_hbm, xg_hbm, idx_v, row0, row1, ld0, ld1, st0, st1):
        wid = lax.axis_index("s") * 2 + lax.axis_index("c")
        base = wid * _JPW
        pltpu.sync_copy(pos_hbm.at[wid], idx_v)
        rows = (row0, row1)
        lds = (ld0, ld1)
        sts = (st0, st1)

        def start_load(c, b):
            pltpu.async_copy(x_hbm.at[pl.ds((base + c * _CH) % T, _CH)],
                             rows[b], lds[b])

        start_load(0, 0)
        start_load(1, 1)

        @pl.loop(0, _NCH, step=2)
        def _(c):
            for b in range(2):
                cc = c + b
                pltpu.make_async_copy(
                    x_hbm.at[pl.ds(0, _CH)], rows[b], lds[b]).wait()
                cp = pltpu.async_copy(rows[b], xg_hbm.at[idx_v.at[cc]],
                                      sts[b])
                cp.wait()

                @pl.when(cc + 2 < _NCH)
                def _():
                    pltpu.async_copy(
                        x_hbm.at[pl.ds((base + (cc + 2) * _CH) % T, _CH)],
                        rows[b], lds[b])

    return k(x, pos3)


def _sc_gather(eout, pos3):
    @functools.partial(
        pl.kernel, mesh=_sc_mesh(),
        out_type=jax.ShapeDtypeStruct((A, D), _F32),
        scratch_types=[pltpu.VMEM((_NCH, _CH), jnp.int32),
                       pltpu.VMEM((_CH, D), _F32),
                       pltpu.VMEM((_CH, D), _F32),
                       pltpu.SemaphoreType.DMA,
                       pltpu.SemaphoreType.DMA,
                       pltpu.SemaphoreType.DMA,
                       pltpu.SemaphoreType.DMA],
    )
    def k(eout_hbm, pos_hbm, eg_hbm, idx_v, row0, row1, ld0, ld1, st0, st1):
        wid = lax.axis_index("s") * 2 + lax.axis_index("c")
        base = wid * _JPW
        pltpu.sync_copy(pos_hbm.at[wid], idx_v)
        rows = (row0, row1)
        lds = (ld0, ld1)
        sts = (st0, st1)

        def start_gather(c, b):
            pltpu.async_copy(eout_hbm.at[idx_v.at[c]], rows[b], lds[b])

        start_gather(0, 0)
        start_gather(1, 1)

        @pl.loop(0, _NCH, step=2)
        def _(c):
            for b in range(2):
                cc = c + b
                pltpu.make_async_copy(
                    eout_hbm.at[pl.ds(0, _CH)], rows[b], lds[b]).wait()
                cp = pltpu.async_copy(
                    rows[b], eg_hbm.at[pl.ds(base + cc * _CH, _CH)], sts[b])
                cp.wait()

                @pl.when(cc + 2 < _NCH)
                def _():
                    start_gather(cc + 2, b)

    return k(eout, pos3)


def kernel(hidden_states, gate_w, expert_gate, expert_up, expert_down,
           shared_gate, shared_up, shared_down):
    b, s, h = hidden_states.shape
    x = hidden_states.reshape(-1, h)

    w2, pos_tk, te, tv, tn, slot, nxte, pf, fi, pi = pl.pallas_call(
        _router_body,
        out_shape=[
            jax.ShapeDtypeStruct((T, TOP_K), _F32),
            jax.ShapeDtypeStruct((T, TOP_K), jnp.int32),
            jax.ShapeDtypeStruct((NT, 1), jnp.int32),
            jax.ShapeDtypeStruct((NT, 1), jnp.int32),
            jax.ShapeDtypeStruct((NT, 1), jnp.int32),
            jax.ShapeDtypeStruct((NT, 1), jnp.int32),
            jax.ShapeDtypeStruct((NT, 1), jnp.int32),
            jax.ShapeDtypeStruct((NT, 1), jnp.int32),
            jax.ShapeDtypeStruct((1, E), _F32),
            jax.ShapeDtypeStruct((1, E), _F32),
        ],
    )(x, gate_w.T)

    pos3 = pos_tk.T.reshape(_NW, _NCH, _CH)
    te_s = te.reshape(NT)
    tv_s = tv.reshape(NT)
    tn_s = tn.reshape(NT)
    slot_s = slot.reshape(NT)
    nxte_s = nxte.reshape(NT)
    pf_s = pf.reshape(NT)


    def _shared_half(off):
        return pl.pallas_call(
            _shared_body,
            grid=(T // 2 // BT,),
            in_specs=[
                pl.BlockSpec((BT, D), lambda t: (t + off, 0)),
                pl.BlockSpec((SH_DFF, D), lambda t: (0, 0)),
                pl.BlockSpec((SH_DFF, D), lambda t: (0, 0)),
                pl.BlockSpec((D, SH_DFF), lambda t: (0, 0)),
            ],
            out_specs=pl.BlockSpec((BT, D), lambda t: (t, 0)),
            out_shape=jax.ShapeDtypeStruct((T // 2, D), _F32),
            compiler_params=pltpu.CompilerParams(
                dimension_semantics=("arbitrary",)),
        )(x, shared_gate, shared_up, shared_down)

    y_sh_a = _shared_half(0)
    xg = _sc_scatter(x, pos3)
    y_sh_b = _shared_half(T // 2 // BT)

    eout = pl.pallas_call(
        _grouped_body,
        grid_spec=pltpu.PrefetchScalarGridSpec(
            num_scalar_prefetch=6,
            grid=(NT,),
            in_specs=[
                pl.BlockSpec((TILE, D), lambda i, *_: (i, 0)),
                pl.BlockSpec(memory_space=pl.ANY),
                pl.BlockSpec(memory_space=pl.ANY),
                pl.BlockSpec(memory_space=pl.ANY),
                pl.BlockSpec((8, 128), lambda i, *_: (0, 0)),
            ],
            out_specs=pl.BlockSpec((TILE, D), lambda i, *_: (i, 0)),
            scratch_shapes=[
                pltpu.VMEM((2, DFF, D), _F32),
                pltpu.VMEM((2, DFF, D), _F32),
                pltpu.VMEM((2, D, DFF), _F32),
                pltpu.SemaphoreType.DMA((2,)),
                pltpu.SemaphoreType.DMA((2,)),
                pltpu.SemaphoreType.DMA((2,)),
            ],
        ),
        out_shape=jax.ShapeDtypeStruct((NPAD, D), _F32),
        compiler_params=pltpu.CompilerParams(
            dimension_semantics=("arbitrary",)),
    )(te_s, tv_s, tn_s, slot_s, nxte_s, pf_s, xg,
      expert_gate, expert_up, expert_down, y_sh_a)

    eg = _sc_gather(eout, pos3).reshape(TOP_K, T, D)

    y = pl.pallas_call(
        _combine_body,
        grid=(T // BT,),
        in_specs=[
            pl.BlockSpec((1, BT, D), lambda t: (0, t, 0)),
            pl.BlockSpec((1, BT, D), lambda t: (1, t, 0)),
            pl.BlockSpec((BT, TOP_K), lambda t: (t, 0)),
            pl.BlockSpec((BT, D),
                         lambda t: (jnp.minimum(t, T // 2 // BT - 1), 0)),
            pl.BlockSpec((BT, D),
                         lambda t: (jnp.maximum(t - T // 2 // BT, 0), 0)),
        ],
        out_specs=pl.BlockSpec((BT, D), lambda t: (t, 0)),
        out_shape=jax.ShapeDtypeStruct((T, D), _F32),
        compiler_params=pltpu.CompilerParams(
            dimension_semantics=("parallel",)),
    )(eg, eg, w2, y_sh_a, y_sh_b)

    return (y.reshape(b, s, h), fi.reshape(E), pi.reshape(E))
